# baseline (speedup 1.0000x reference)
import functools

import jax
import jax.numpy as jnp
from jax import lax
from jax.experimental import pallas as pl
from jax.experimental.pallas import tpu as pltpu
from jax.experimental.pallas import tpu_sc as plsc

B, N, M = 2, 2048, 8192
C_IN, C1, C2 = 64, 64, 64
K = 3
TM = 256
NB = M // TM
BM = B * M

NCORES = 2
NSUB = 16
NW = NCORES * NSUB
QPW = BM // NW
CH = 256


_FBIG = 3e38


def _knn_body(qT_ref, p_ref, i0_ref, i1_ref, i2_ref, w0_ref, w1_ref, w2_ref):
    b = pl.program_id(0)
    qt = qT_ref[0]
    qx, qy, qz = qt[0:1, :], qt[1:2, :], qt[2:3, :]
    pt = p_ref[0]
    px, py, pz = pt[:, 0:1], pt[:, 1:2], pt[:, 2:3]
    dx = px - qx
    dy = py - qy
    dz = pz - qz
    d = dx * dx + dy * dy + dz * dz
    iota = lax.broadcasted_iota(jnp.int32, (N, TM), 0)

    m1 = jnp.min(d, axis=0, keepdims=True)
    i1 = jnp.min(jnp.where(d == m1, iota, N), axis=0, keepdims=True)
    c2 = jnp.where(iota == i1, _FBIG, d)
    m2 = jnp.min(c2, axis=0, keepdims=True)
    i2 = jnp.min(jnp.where(c2 == m2, iota, N), axis=0, keepdims=True)
    c3 = jnp.where(iota == i2, _FBIG, c2)
    m3 = jnp.min(c3, axis=0, keepdims=True)
    i3 = jnp.min(jnp.where(c3 == m3, iota, N), axis=0, keepdims=True)

    ws = [1.0 / jnp.maximum(v, 1e-10) for v in (m1, m2, m3)]
    wsum = ws[0] + ws[1] + ws[2]
    base = b * N
    for i_ref, w_ref, i, w in zip((i0_ref, i1_ref, i2_ref),
                                  (w0_ref, w1_ref, w2_ref), (i1, i2, i3), ws):
        i_ref[...] = (i + base)[None]
        w_ref[...] = (w / wsum)[None]


def _sc_gather_body(idx0, idx1, idx2, xT, out0, out1, out2,
                    idx0_v, idx1_v, idx2_v,
                    rows0_v, rows1_v, rows2_v, sem0, sem1, sem2):
    wid = lax.axis_index("s") * NCORES + lax.axis_index("c")
    qb = wid * QPW
    pltpu.sync_copy(idx0.at[pl.ds(qb, QPW)], idx0_v)
    pltpu.sync_copy(idx1.at[pl.ds(qb, QPW)], idx1_v)
    pltpu.sync_copy(idx2.at[pl.ds(qb, QPW)], idx2_v)
    cp0 = pltpu.async_copy(xT.at[idx0_v], rows0_v, sem0)
    cp1 = pltpu.async_copy(xT.at[idx1_v], rows1_v, sem1)
    cp2 = pltpu.async_copy(xT.at[idx2_v], rows2_v, sem2)
    cp0.wait()
    pltpu.sync_copy(rows0_v, out0.at[pl.ds(qb, QPW)])
    cp1.wait()
    pltpu.sync_copy(rows1_v, out1.at[pl.ds(qb, QPW)])
    cp2.wait()
    pltpu.sync_copy(rows2_v, out2.at[pl.ds(qb, QPW)])


TR = 2048
NT = BM // TR


def _accum_stats(u, s_ref, q_ref):
    ps = jnp.sum(u, axis=0, keepdims=True)
    pq = jnp.sum(u * u, axis=0, keepdims=True)

    @pl.when(pl.program_id(0) == 0)
    def _():
        s_ref[...] = jnp.zeros_like(s_ref)
        q_ref[...] = jnp.zeros_like(q_ref)

    s_ref[...] += ps
    q_ref[...] += pq


def _bn_from_stats(u, s_ref, q_ref, g_ref, be_ref):
    mu = s_ref[...] * (1.0 / BM)
    var = q_ref[...] * (1.0 / BM) - mu * mu
    r = (u - mu) * lax.rsqrt(var + 1e-5) * g_ref[...] + be_ref[...]
    return jnp.maximum(r, 0.0)


def _mlp1_body(r0_ref, r1_ref, r2_ref, w0_ref, w1_ref, w2_ref,
               W1_ref, b1_ref, u1_ref, s1_ref, q1_ref):
    h = (r0_ref[...] * w0_ref[...] + r1_ref[...] * w1_ref[...]
         + r2_ref[...] * w2_ref[...])
    u = lax.dot_general(h, W1_ref[...], (((1,), (1,)), ((), ())),
                        preferred_element_type=jnp.float32) + b1_ref[...]
    u1_ref[...] = u
    _accum_stats(u, s1_ref, q1_ref)


def _mlp2_body(u1_ref, s1_ref, q1_ref, g1_ref, be1_ref, W2_ref, b2_ref,
               u2_ref, s2_ref, q2_ref):
    r = _bn_from_stats(u1_ref[...], s1_ref, q1_ref, g1_ref, be1_ref)
    u = lax.dot_general(r, W2_ref[...], (((1,), (1,)), ((), ())),
                        preferred_element_type=jnp.float32) + b2_ref[...]
    u2_ref[...] = u
    _accum_stats(u, s2_ref, q2_ref)


def _mlp3_body(u2_ref, s2_ref, q2_ref, g2_ref, be2_ref, out_ref):
    out_ref[...] = _bn_from_stats(u2_ref[...], s2_ref, q2_ref,
                                  g2_ref, be2_ref)


def _knn_call(qT, p):
    iw_spec = pl.BlockSpec((1, 1, TM), lambda b, i: (b * NB + i, 0, 0))
    return pl.pallas_call(
        _knn_body,
        grid=(B, NB),
        in_specs=[
            pl.BlockSpec((1, 3, TM), lambda b, i: (b, 0, i)),
            pl.BlockSpec((1, N, 3), lambda b, i: (b, 0, 0)),
        ],
        out_specs=[iw_spec] * 6,
        out_shape=[jax.ShapeDtypeStruct((B * NB, 1, TM), jnp.int32)] * 3
        + [jax.ShapeDtypeStruct((B * NB, 1, TM), jnp.float32)] * 3,
    )(qT, p)


@functools.lru_cache(maxsize=1)
def _build_sc_gather():
    @functools.partial(
        pl.kernel,
        out_type=[jax.ShapeDtypeStruct((BM, C_IN), jnp.float32)] * 3,
        mesh=plsc.VectorSubcoreMesh(core_axis_name="c", subcore_axis_name="s"),
        compiler_params=pltpu.CompilerParams(use_tc_tiling_on_sc=False),
        scratch_types=[
            pltpu.VMEM((QPW,), jnp.int32),
            pltpu.VMEM((QPW,), jnp.int32),
            pltpu.VMEM((QPW,), jnp.int32),
            pltpu.VMEM((QPW, C_IN), jnp.float32),
            pltpu.VMEM((QPW, C_IN), jnp.float32),
            pltpu.VMEM((QPW, C_IN), jnp.float32),
            pltpu.SemaphoreType.DMA,
            pltpu.SemaphoreType.DMA,
            pltpu.SemaphoreType.DMA,
        ],
    )
    def _sc_gather(idx0, idx1, idx2, xT, out0, out1, out2, *scratch):
        _sc_gather_body(idx0, idx1, idx2, xT, out0, out1, out2, *scratch)

    return _sc_gather


def kernel(p, q, x, W1, b1, g1, be1, W2, b2, g2, be2):
    qT = jnp.swapaxes(q, 1, 2)
    xTf = jnp.swapaxes(x, 1, 2).reshape(B * N, C_IN)

    i0, i1, i2, w0, w1, w2 = _knn_call(qT, p)
    w0, w1, w2 = (w.reshape(BM, 1) for w in (w0, w1, w2))
    r0, r1, r2 = _build_sc_gather()(i0.reshape(BM), i1.reshape(BM),
                                    i2.reshape(BM), xTf)

    row = pl.BlockSpec((TR, C1), lambda i: (i, 0))
    w_spec = pl.BlockSpec((TR, 1), lambda i: (i, 0))
    vec = pl.BlockSpec((1, C1), lambda i: (0, 0))
    mat = pl.BlockSpec((C1, C1), lambda i: (0, 0))
    stat_shape = jax.ShapeDtypeStruct((1, C1), jnp.float32)
    row_shape = jax.ShapeDtypeStruct((BM, C1), jnp.float32)

    u1, s1, q1 = pl.pallas_call(
        _mlp1_body, grid=(NT,),
        in_specs=[row, row, row, w_spec, w_spec, w_spec, mat, vec],
        out_specs=[row, vec, vec],
        out_shape=[row_shape, stat_shape, stat_shape],
    )(r0, r1, r2, w0, w1, w2, W1, b1[None, :])

    u2, s2, q2 = pl.pallas_call(
        _mlp2_body, grid=(NT,),
        in_specs=[row, vec, vec, vec, vec, mat, vec],
        out_specs=[row, vec, vec],
        out_shape=[row_shape, stat_shape, stat_shape],
    )(u1, s1, q1, g1[None, :], be1[None, :], W2, b2[None, :])

    out = pl.pallas_call(
        _mlp3_body, grid=(NT,),
        in_specs=[row, vec, vec, vec, vec],
        out_specs=row,
        out_shape=jax.ShapeDtypeStruct((BM, C2), jnp.float32),
    )(u2, s2, q2, g2[None, :], be2[None, :])

    h = jnp.swapaxes(out.reshape(B, M, C2), 1, 2)
    return (q, h)


# speedup vs baseline: 1.2393x; 1.2393x over previous
import functools

import jax
import jax.numpy as jnp
from jax import lax
from jax.experimental import pallas as pl
from jax.experimental.pallas import tpu as pltpu
from jax.experimental.pallas import tpu_sc as plsc

B, N, M = 2, 2048, 8192
C_IN, C1, C2 = 64, 64, 64
K = 3
TM = 256
NB = M // TM
BM = B * M

NCORES = 2
NSUB = 16
NW = NCORES * NSUB
QPW = BM // NW
CH = 256


_FBIG = 3e38
_G = 32
_O2 = 4096.0
_O3 = 8192.0


def _ins3(a, b, c, v):
    a2 = jnp.minimum(a, v)
    t = jnp.maximum(a, v)
    b2 = jnp.minimum(b, t)
    u = jnp.maximum(b, t)
    c2 = jnp.minimum(c, u)
    return a2, b2, c2


def _fold3(a, b, c):
    g = a.shape[0]
    while g > 1:
        h = g // 2
        a1, b1, c1 = a[:h], b[:h], c[:h]
        for v in (a[h:], b[h:], c[h:]):
            a1, b1, c1 = _ins3(a1, b1, c1, v)
        a, b, c = a1, b1, c1
        g = h
    return a, b, c


def _knn_body(qT_ref, p_ref, i0_ref, i1_ref, i2_ref, w0_ref, w1_ref, w2_ref):
    b = pl.program_id(0)
    qt = qT_ref[0]
    qx, qy, qz = qt[0:1, :], qt[1:2, :], qt[2:3, :]
    pt = p_ref[0]
    px, py, pz = pt[:, 0:1], pt[:, 1:2], pt[:, 2:3]
    dx = px - qx
    dy = py - qy
    dz = pz - qz
    d = dx * dx + dy * dy + dz * dz

    full = functools.partial(jnp.full, (_G, TM), dtype=jnp.float32)
    a, bb, c = full(_FBIG), full(_FBIG), full(_FBIG)
    for s in range(N // _G):
        a, bb, c = _ins3(a, bb, c, d[s * _G:(s + 1) * _G])
    m1, m2, m3 = _fold3(a, bb, c)

    io = lax.broadcasted_iota(jnp.int32, (_G, TM), 0).astype(jnp.float32)
    ja, jb, jc = full(_FBIG), full(_FBIG), full(_FBIG)
    for s in range(N // _G):
        ds = d[s * _G:(s + 1) * _G]
        ios = io + float(s * _G)
        ks = jnp.where(ds == m1, ios,
                       jnp.where(ds == m2, ios + _O2,
                                 jnp.where(ds == m3, ios + _O3, _FBIG)))
        ja, jb, jc = _ins3(ja, jb, jc, ks)
    k1, k2, k3 = _fold3(ja, jb, jc)

    def deco(k):
        return k - jnp.where(k >= _O3, _O3, jnp.where(k >= _O2, _O2, 0.0))

    idxs = [jnp.int32(deco(k)) for k in (k1, k2, k3)]
    ws = [1.0 / jnp.maximum(v, 1e-10) for v in (m1, m2, m3)]
    wsum = ws[0] + ws[1] + ws[2]
    base = b * N
    for i_ref, w_ref, i, w in zip((i0_ref, i1_ref, i2_ref),
                                  (w0_ref, w1_ref, w2_ref), idxs, ws):
        i_ref[...] = (i + base)[None]
        w_ref[...] = (w / wsum)[None]


def _sc_gather_body(idx0, idx1, idx2, xT, out0, out1, out2,
                    idx0_v, idx1_v, idx2_v,
                    rows0_v, rows1_v, rows2_v, sem0, sem1, sem2):
    wid = lax.axis_index("s") * NCORES + lax.axis_index("c")
    qb = wid * QPW
    pltpu.sync_copy(idx0.at[pl.ds(qb, QPW)], idx0_v)
    pltpu.sync_copy(idx1.at[pl.ds(qb, QPW)], idx1_v)
    pltpu.sync_copy(idx2.at[pl.ds(qb, QPW)], idx2_v)
    cp0 = pltpu.async_copy(xT.at[idx0_v], rows0_v, sem0)
    cp1 = pltpu.async_copy(xT.at[idx1_v], rows1_v, sem1)
    cp2 = pltpu.async_copy(xT.at[idx2_v], rows2_v, sem2)
    cp0.wait()
    pltpu.sync_copy(rows0_v, out0.at[pl.ds(qb, QPW)])
    cp1.wait()
    pltpu.sync_copy(rows1_v, out1.at[pl.ds(qb, QPW)])
    cp2.wait()
    pltpu.sync_copy(rows2_v, out2.at[pl.ds(qb, QPW)])


TR = 2048
NT = BM // TR
MT = M // TR


def _bn_cm(u, s_ref, q_ref, g_ref, be_ref):
    mu = s_ref[...] * (1.0 / BM)
    var = q_ref[...] * (1.0 / BM) - mu * mu
    r = (u - mu) * lax.rsqrt(var + 1e-5) * g_ref[...] + be_ref[...]
    return jnp.maximum(r, 0.0)


def _mlp_body(r0, r1, r2, w0, w1, w2, W1, b1, g1, be1, W2, b2, g2, be2,
              out_ref, u1s, u2s, s1, q1, s2, q2):
    p = pl.program_id(0)
    i = pl.program_id(1)
    sl = pl.ds(i * TR, TR)

    @pl.when(jnp.logical_and(p == 0, i == 0))
    def _():
        for r in (s1, q1, s2, q2):
            r[...] = jnp.zeros_like(r)

    @pl.when(p == 0)
    def _():
        h = (r0[...] * w0[...] + r1[...] * w1[...] + r2[...] * w2[...])
        u = lax.dot_general(W1[...], h, (((1,), (1,)), ((), ())),
                            preferred_element_type=jnp.float32) + b1[...]
        u1s[:, sl] = u
        s1[...] += jnp.sum(u, axis=1, keepdims=True)
        q1[...] += jnp.sum(u * u, axis=1, keepdims=True)

    @pl.when(p == 1)
    def _():
        r = _bn_cm(u1s[:, sl], s1, q1, g1, be1)
        u = lax.dot_general(W2[...], r, (((1,), (0,)), ((), ())),
                            preferred_element_type=jnp.float32) + b2[...]
        u2s[:, sl] = u
        s2[...] += jnp.sum(u, axis=1, keepdims=True)
        q2[...] += jnp.sum(u * u, axis=1, keepdims=True)

    @pl.when(p == 2)
    def _():
        out_ref[0] = _bn_cm(u2s[:, sl], s2, q2, g2, be2)


def _knn_call(qT, p):
    iw_spec = pl.BlockSpec((1, 1, TM), lambda b, i: (b * NB + i, 0, 0))
    return pl.pallas_call(
        _knn_body,
        grid=(B, NB),
        in_specs=[
            pl.BlockSpec((1, 3, TM), lambda b, i: (b, 0, i)),
            pl.BlockSpec((1, N, 3), lambda b, i: (b, 0, 0)),
        ],
        out_specs=[iw_spec] * 6,
        out_shape=[jax.ShapeDtypeStruct((B * NB, 1, TM), jnp.int32)] * 3
        + [jax.ShapeDtypeStruct((B * NB, 1, TM), jnp.float32)] * 3,
    )(qT, p)


@functools.lru_cache(maxsize=1)
def _build_sc_gather():
    @functools.partial(
        pl.kernel,
        out_type=[jax.ShapeDtypeStruct((BM, C_IN), jnp.float32)] * 3,
        mesh=plsc.VectorSubcoreMesh(core_axis_name="c", subcore_axis_name="s"),
        compiler_params=pltpu.CompilerParams(use_tc_tiling_on_sc=False),
        scratch_types=[
            pltpu.VMEM((QPW,), jnp.int32),
            pltpu.VMEM((QPW,), jnp.int32),
            pltpu.VMEM((QPW,), jnp.int32),
            pltpu.VMEM((QPW, C_IN), jnp.float32),
            pltpu.VMEM((QPW, C_IN), jnp.float32),
            pltpu.VMEM((QPW, C_IN), jnp.float32),
            pltpu.SemaphoreType.DMA,
            pltpu.SemaphoreType.DMA,
            pltpu.SemaphoreType.DMA,
        ],
    )
    def _sc_gather(idx0, idx1, idx2, xT, out0, out1, out2, *scratch):
        _sc_gather_body(idx0, idx1, idx2, xT, out0, out1, out2, *scratch)

    return _sc_gather


def kernel(p, q, x, W1, b1, g1, be1, W2, b2, g2, be2):
    qT = jnp.swapaxes(q, 1, 2)
    xTf = jnp.swapaxes(x, 1, 2).reshape(B * N, C_IN)

    i0, i1, i2, w0, w1, w2 = _knn_call(qT, p)
    w0, w1, w2 = (w.reshape(BM, 1) for w in (w0, w1, w2))
    r0, r1, r2 = _build_sc_gather()(i0.reshape(BM), i1.reshape(BM),
                                    i2.reshape(BM), xTf)

    row = pl.BlockSpec((TR, C1), lambda p, i: (jnp.where(p == 0, i, 0), 0))
    w_spec = pl.BlockSpec((TR, 1), lambda p, i: (jnp.where(p == 0, i, 0), 0))
    vec = pl.BlockSpec((C1, 1), lambda p, i: (0, 0))
    mat = pl.BlockSpec((C1, C1), lambda p, i: (0, 0))

    h = pl.pallas_call(
        _mlp_body, grid=(3, NT),
        in_specs=[row, row, row, w_spec, w_spec, w_spec,
                  mat, vec, vec, vec, mat, vec, vec, vec],
        out_specs=pl.BlockSpec(
            (1, C2, TR),
            lambda p, i: (jnp.where(p == 2, i // MT, 0), 0,
                          jnp.where(p == 2, i % MT, 0))),
        out_shape=jax.ShapeDtypeStruct((B, C2, M), jnp.float32),
        scratch_shapes=[
            pltpu.VMEM((C1, BM), jnp.float32),
            pltpu.VMEM((C2, BM), jnp.float32),
            pltpu.VMEM((C1, 1), jnp.float32),
            pltpu.VMEM((C1, 1), jnp.float32),
            pltpu.VMEM((C2, 1), jnp.float32),
            pltpu.VMEM((C2, 1), jnp.float32),
        ],
    )(r0, r1, r2, w0, w1, w2, W1, b1[:, None], g1[:, None], be1[:, None],
      W2, b2[:, None], g2[:, None], be2[:, None])

    return (q, h)


# speedup vs baseline: 1.3575x; 1.0954x over previous
import functools

import jax
import jax.numpy as jnp
from jax import lax
from jax.experimental import pallas as pl
from jax.experimental.pallas import tpu as pltpu
from jax.experimental.pallas import tpu_sc as plsc

B, N, M = 2, 2048, 8192
C_IN, C1, C2 = 64, 64, 64
K = 3
TM = 512
NB = M // TM
BM = B * M

NCORES = 2
NSUB = 16
NW = NCORES * NSUB
QPW = BM // NW
CH = 256


_FBIG = 3e38
_G = 16
_O2 = 4096.0
_O3 = 8192.0


def _ins3(a, b, c, v):
    a2 = jnp.minimum(a, v)
    t = jnp.maximum(a, v)
    b2 = jnp.minimum(b, t)
    u = jnp.maximum(b, t)
    c2 = jnp.minimum(c, u)
    return a2, b2, c2


def _fold3(a, b, c):
    g = a.shape[0]
    while g > 1:
        h = g // 2
        a1, b1, c1 = a[:h], b[:h], c[:h]
        for v in (a[h:], b[h:], c[h:]):
            a1, b1, c1 = _ins3(a1, b1, c1, v)
        a, b, c = a1, b1, c1
        g = h
    return a, b, c


def _knn_body(qT_ref, p_ref, i0_ref, i1_ref, i2_ref, w0_ref, w1_ref, w2_ref):
    b = pl.program_id(0)
    qt = qT_ref[0]
    qx, qy, qz = qt[0:1, :], qt[1:2, :], qt[2:3, :]
    pt = p_ref[0]
    px, py, pz = pt[:, 0:1], pt[:, 1:2], pt[:, 2:3]
    dx = px - qx
    dy = py - qy
    dz = pz - qz
    d = dx * dx + dy * dy + dz * dz

    full = functools.partial(jnp.full, (_G, TM), dtype=jnp.float32)
    a, bb, c = full(_FBIG), full(_FBIG), full(_FBIG)
    for s in range(N // _G):
        a, bb, c = _ins3(a, bb, c, d[s * _G:(s + 1) * _G])
    m1, m2, m3 = _fold3(a, bb, c)

    io = lax.broadcasted_iota(jnp.int32, (_G, TM), 0).astype(jnp.float32)
    ja, jb, jc = full(_FBIG), full(_FBIG), full(_FBIG)
    for s in range(N // _G):
        ds = d[s * _G:(s + 1) * _G]
        ios = io + float(s * _G)
        cls = jnp.where(ds == m1, 0.0,
                        jnp.where(ds == m2, _O2,
                                  jnp.where(ds == m3, _O3, _FBIG)))
        ks = ios + cls
        ja, jb, jc = _ins3(ja, jb, jc, ks)
    k1, k2, k3 = _fold3(ja, jb, jc)

    def deco(k):
        return k - jnp.where(k >= _O3, _O3, jnp.where(k >= _O2, _O2, 0.0))

    idxs = [jnp.int32(deco(k)) for k in (k1, k2, k3)]
    ws = [1.0 / jnp.maximum(v, 1e-10) for v in (m1, m2, m3)]
    wsum = ws[0] + ws[1] + ws[2]
    base = b * N
    for i_ref, w_ref, i, w in zip((i0_ref, i1_ref, i2_ref),
                                  (w0_ref, w1_ref, w2_ref), idxs, ws):
        i_ref[...] = (i + base)[None]
        w_ref[...] = (w / wsum)[None]


NCH = QPW // CH


def _sc_gather_body(idx0, idx1, idx2, xT, out0, out1, out2, *scr):
    nj = 3 * NCH
    idx_v = scr[0:nj]
    rows_v = scr[nj:2 * nj]
    isems = scr[2 * nj:3 * nj]
    osems = scr[3 * nj:4 * nj]
    ins = (idx0, idx1, idx2)
    outs = (out0, out1, out2)
    wid = lax.axis_index("s") * NCORES + lax.axis_index("c")
    qb = wid * QPW
    gops = []
    for ch in range(NCH):
        cb = qb + ch * CH
        for k in range(3):
            j = ch * 3 + k
            pltpu.sync_copy(ins[k].at[pl.ds(cb, CH)], idx_v[j])
            gops.append(pltpu.async_copy(xT.at[idx_v[j]], rows_v[j],
                                         isems[j]))
    oops = []
    for ch in range(NCH):
        cb = qb + ch * CH
        for k in range(3):
            j = ch * 3 + k
            gops[j].wait()
            oops.append(pltpu.async_copy(rows_v[j], outs[k].at[pl.ds(cb, CH)],
                                         osems[j]))
    for op in oops:
        op.wait()


TR = 4096
NT = BM // TR
MT = M // TR


def _bn_cm(u, s_ref, q_ref, g_ref, be_ref):
    mu = s_ref[...] * (1.0 / BM)
    var = q_ref[...] * (1.0 / BM) - mu * mu
    r = (u - mu) * lax.rsqrt(var + 1e-5) * g_ref[...] + be_ref[...]
    return jnp.maximum(r, 0.0)


def _mlp_body(r0, r1, r2, w0, w1, w2, W1, b1, g1, be1, W2, b2, g2, be2,
              out_ref, u1s, u2s, s1, q1, s2, q2):
    p = pl.program_id(0)
    i = pl.program_id(1)
    sl = pl.ds(i * TR, TR)

    @pl.when(jnp.logical_and(p == 0, i == 0))
    def _():
        for r in (s1, q1, s2, q2):
            r[...] = jnp.zeros_like(r)

    @pl.when(p == 0)
    def _():
        h = (r0[...] * w0[...] + r1[...] * w1[...] + r2[...] * w2[...])
        u = lax.dot_general(W1[...], h, (((1,), (1,)), ((), ())),
                            preferred_element_type=jnp.float32) + b1[...]
        u1s[:, sl] = u
        s1[...] += jnp.sum(u, axis=1, keepdims=True)
        q1[...] += jnp.sum(u * u, axis=1, keepdims=True)

    @pl.when(p == 1)
    def _():
        r = _bn_cm(u1s[:, sl], s1, q1, g1, be1)
        u = lax.dot_general(W2[...], r, (((1,), (0,)), ((), ())),
                            preferred_element_type=jnp.float32) + b2[...]
        u2s[:, sl] = u
        s2[...] += jnp.sum(u, axis=1, keepdims=True)
        q2[...] += jnp.sum(u * u, axis=1, keepdims=True)

    @pl.when(p == 2)
    def _():
        out_ref[0] = _bn_cm(u2s[:, sl], s2, q2, g2, be2)


def _knn_call(qT, p):
    iw_spec = pl.BlockSpec((1, 1, TM), lambda b, i: (b * NB + i, 0, 0))
    return pl.pallas_call(
        _knn_body,
        grid=(B, NB),
        in_specs=[
            pl.BlockSpec((1, 3, TM), lambda b, i: (b, 0, i)),
            pl.BlockSpec((1, N, 3), lambda b, i: (b, 0, 0)),
        ],
        out_specs=[iw_spec] * 6,
        out_shape=[jax.ShapeDtypeStruct((B * NB, 1, TM), jnp.int32)] * 3
        + [jax.ShapeDtypeStruct((B * NB, 1, TM), jnp.float32)] * 3,
    )(qT, p)


@functools.lru_cache(maxsize=1)
def _build_sc_gather():
    @functools.partial(
        pl.kernel,
        out_type=[jax.ShapeDtypeStruct((BM, C_IN), jnp.float32)] * 3,
        mesh=plsc.VectorSubcoreMesh(core_axis_name="c", subcore_axis_name="s"),
        compiler_params=pltpu.CompilerParams(use_tc_tiling_on_sc=False),
        scratch_types=(
            [pltpu.VMEM((CH,), jnp.int32)] * (3 * NCH)
            + [pltpu.VMEM((CH, C_IN), jnp.float32)] * (3 * NCH)
            + [pltpu.SemaphoreType.DMA] * (6 * NCH)
        ),
    )
    def _sc_gather(idx0, idx1, idx2, xT, out0, out1, out2, *scratch):
        _sc_gather_body(idx0, idx1, idx2, xT, out0, out1, out2, *scratch)

    return _sc_gather


def kernel(p, q, x, W1, b1, g1, be1, W2, b2, g2, be2):
    qT = jnp.swapaxes(q, 1, 2)
    xTf = jnp.swapaxes(x, 1, 2).reshape(B * N, C_IN)

    i0, i1, i2, w0, w1, w2 = _knn_call(qT, p)
    w0, w1, w2 = (w.reshape(BM, 1) for w in (w0, w1, w2))
    r0, r1, r2 = _build_sc_gather()(i0.reshape(BM), i1.reshape(BM),
                                    i2.reshape(BM), xTf)

    row = pl.BlockSpec((TR, C1), lambda p, i: (jnp.where(p == 0, i, 0), 0))
    w_spec = pl.BlockSpec((TR, 1), lambda p, i: (jnp.where(p == 0, i, 0), 0))
    vec = pl.BlockSpec((C1, 1), lambda p, i: (0, 0))
    mat = pl.BlockSpec((C1, C1), lambda p, i: (0, 0))

    h = pl.pallas_call(
        _mlp_body, grid=(3, NT),
        in_specs=[row, row, row, w_spec, w_spec, w_spec,
                  mat, vec, vec, vec, mat, vec, vec, vec],
        out_specs=pl.BlockSpec(
            (1, C2, TR),
            lambda p, i: (jnp.where(p == 2, i // MT, 0), 0,
                          jnp.where(p == 2, i % MT, 0))),
        out_shape=jax.ShapeDtypeStruct((B, C2, M), jnp.float32),
        scratch_shapes=[
            pltpu.VMEM((C1, BM), jnp.float32),
            pltpu.VMEM((C2, BM), jnp.float32),
            pltpu.VMEM((C1, 1), jnp.float32),
            pltpu.VMEM((C1, 1), jnp.float32),
            pltpu.VMEM((C2, 1), jnp.float32),
            pltpu.VMEM((C2, 1), jnp.float32),
        ],
    )(r0, r1, r2, w0, w1, w2, W1, b1[:, None], g1[:, None], be1[:, None],
      W2, b2[:, None], g2[:, None], be2[:, None])

    return (q, h)


# speedup vs baseline: 1.6454x; 1.2121x over previous
import functools

import jax
import jax.numpy as jnp
from jax import lax
from jax.experimental import pallas as pl
from jax.experimental.pallas import tpu as pltpu
from jax.experimental.pallas import tpu_sc as plsc

B, N, M = 2, 2048, 8192
C_IN, C1, C2 = 64, 64, 64
K = 3
TM = 512
NB = M // TM
BM = B * M

NCORES = 2
NSUB = 16
NW = NCORES * NSUB
QPW = BM // NW
CH = 256


_FBIG = 3e38
_G = 16
_O2 = 4096.0
_O3 = 8192.0


def _ins3(a, b, c, v):
    a2 = jnp.minimum(a, v)
    t = jnp.maximum(a, v)
    b2 = jnp.minimum(b, t)
    u = jnp.maximum(b, t)
    c2 = jnp.minimum(c, u)
    return a2, b2, c2


def _fold3(a, b, c):
    g = a.shape[0]
    while g > 1:
        h = g // 2
        a1, b1, c1 = a[:h], b[:h], c[:h]
        for v in (a[h:], b[h:], c[h:]):
            a1, b1, c1 = _ins3(a1, b1, c1, v)
        a, b, c = a1, b1, c1
        g = h
    return a, b, c


def _knn_body(qT_ref, p_ref, i0_ref, i1_ref, i2_ref, w0_ref, w1_ref, w2_ref):
    b = pl.program_id(0)
    qt = qT_ref[0]
    qx, qy, qz = qt[0:1, :], qt[1:2, :], qt[2:3, :]
    pt = p_ref[0]
    px, py, pz = pt[:, 0:1], pt[:, 1:2], pt[:, 2:3]
    dx = px - qx
    dy = py - qy
    dz = pz - qz
    d = dx * dx + dy * dy + dz * dz

    full = functools.partial(jnp.full, (_G, TM), dtype=jnp.float32)
    a, bb, c = full(_FBIG), full(_FBIG), full(_FBIG)
    for s in range(N // _G):
        a, bb, c = _ins3(a, bb, c, d[s * _G:(s + 1) * _G])
    m1, m2, m3 = _fold3(a, bb, c)

    io = lax.broadcasted_iota(jnp.int32, (_G, TM), 0).astype(jnp.float32)
    ja, jb, jc = full(_FBIG), full(_FBIG), full(_FBIG)
    for s in range(N // _G):
        ds = d[s * _G:(s + 1) * _G]
        ios = io + float(s * _G)
        cls = jnp.where(ds == m1, 0.0,
                        jnp.where(ds == m2, _O2,
                                  jnp.where(ds == m3, _O3, _FBIG)))
        ks = ios + cls
        ja, jb, jc = _ins3(ja, jb, jc, ks)
    k1, k2, k3 = _fold3(ja, jb, jc)

    def deco(k):
        return k - jnp.where(k >= _O3, _O3, jnp.where(k >= _O2, _O2, 0.0))

    idxs = [jnp.int32(deco(k)) for k in (k1, k2, k3)]
    ws = [1.0 / jnp.maximum(v, 1e-10) for v in (m1, m2, m3)]
    wsum = ws[0] + ws[1] + ws[2]
    base = b * N
    for i_ref, w_ref, i, w in zip((i0_ref, i1_ref, i2_ref),
                                  (w0_ref, w1_ref, w2_ref), idxs, ws):
        i_ref[...] = (i + base)[None]
        w_ref[...] = (w / wsum)[None]


NCH = QPW // CH


def _sc_gather_body(idx0, idx1, idx2, xT, out0, out1, out2, *scr):
    nj = 3 * NCH
    idx_v = scr[0:nj]
    rows_v = scr[nj:2 * nj]
    isems = scr[2 * nj:3 * nj]
    osems = scr[3 * nj:4 * nj]
    ins = (idx0, idx1, idx2)
    outs = (out0, out1, out2)
    wid = lax.axis_index("s") * NCORES + lax.axis_index("c")
    qb = wid * QPW
    gops = []
    for ch in range(NCH):
        cb = qb + ch * CH
        for k in range(3):
            j = ch * 3 + k
            pltpu.sync_copy(ins[k].at[pl.ds(cb, CH)], idx_v[j])
            gops.append(pltpu.async_copy(xT.at[idx_v[j]], rows_v[j],
                                         isems[j]))
    oops = []
    for ch in range(NCH):
        cb = qb + ch * CH
        for k in range(3):
            j = ch * 3 + k
            gops[j].wait()
            oops.append(pltpu.async_copy(
                rows_v[j], outs[k].at[pl.ds(cb, CH), pl.ds(0, C_IN)],
                osems[j]))
    for op in oops:
        op.wait()


TR = 4096
NT = BM // TR
MT = M // TR


def _bn_cm(u, s_ref, q_ref, g_ref, be_ref):
    mu = s_ref[...] * (1.0 / BM)
    var = q_ref[...] * (1.0 / BM) - mu * mu
    r = (u - mu) * lax.rsqrt(var + 1e-5) * g_ref[...] + be_ref[...]
    return jnp.maximum(r, 0.0)


def _mlp_body(r0, r1, r2, w0, w1, w2, W1, b1, g1, be1, W2, b2, g2, be2,
              out_ref, u1s, u2s, s1, q1, s2, q2):
    p = pl.program_id(0)
    i = pl.program_id(1)
    sl = pl.ds(i * TR, TR)

    @pl.when(jnp.logical_and(p == 0, i == 0))
    def _():
        for r in (s1, q1, s2, q2):
            r[...] = jnp.zeros_like(r)

    @pl.when(p == 0)
    def _():
        Wm = W1[...]
        vs = []
        for r in (r0, r1, r2):
            vs.append(lax.dot_general(Wm, r[...][:, 0:C_IN],
                                      (((1,), (1,)), ((), ())),
                                      preferred_element_type=jnp.float32))
        u = (vs[0] * w0[0] + vs[1] * w1[0] + vs[2] * w2[0]) + b1[...]
        u1s[:, sl] = u
        s1[...] += jnp.sum(u, axis=1, keepdims=True)
        q1[...] += jnp.sum(u * u, axis=1, keepdims=True)

    @pl.when(p == 1)
    def _():
        r = _bn_cm(u1s[:, sl], s1, q1, g1, be1)
        u = lax.dot_general(W2[...], r, (((1,), (0,)), ((), ())),
                            preferred_element_type=jnp.float32) + b2[...]
        u2s[:, sl] = u
        s2[...] += jnp.sum(u, axis=1, keepdims=True)
        q2[...] += jnp.sum(u * u, axis=1, keepdims=True)

    @pl.when(p == 2)
    def _():
        out_ref[0] = _bn_cm(u2s[:, sl], s2, q2, g2, be2)


def _knn_call(qT, p):
    iw_spec = pl.BlockSpec((1, 1, TM), lambda b, i: (b * NB + i, 0, 0))
    return pl.pallas_call(
        _knn_body,
        grid=(B, NB),
        in_specs=[
            pl.BlockSpec((1, 3, TM), lambda b, i: (b, 0, i)),
            pl.BlockSpec((1, N, 3), lambda b, i: (b, 0, 0)),
        ],
        out_specs=[iw_spec] * 6,
        out_shape=[jax.ShapeDtypeStruct((B * NB, 1, TM), jnp.int32)] * 3
        + [jax.ShapeDtypeStruct((B * NB, 1, TM), jnp.float32)] * 3,
    )(qT, p)


@functools.lru_cache(maxsize=1)
def _build_sc_gather():
    @functools.partial(
        pl.kernel,
        out_type=[jax.ShapeDtypeStruct((BM, 128), jnp.float32)] * 3,
        mesh=plsc.VectorSubcoreMesh(core_axis_name="c", subcore_axis_name="s"),
        compiler_params=pltpu.CompilerParams(use_tc_tiling_on_sc=False),
        scratch_types=(
            [pltpu.VMEM((CH,), jnp.int32)] * (3 * NCH)
            + [pltpu.VMEM((CH, C_IN), jnp.float32)] * (3 * NCH)
            + [pltpu.SemaphoreType.DMA] * (6 * NCH)
        ),
    )
    def _sc_gather(idx0, idx1, idx2, xT, out0, out1, out2, *scratch):
        _sc_gather_body(idx0, idx1, idx2, xT, out0, out1, out2, *scratch)

    return _sc_gather


def kernel(p, q, x, W1, b1, g1, be1, W2, b2, g2, be2):
    qT = jnp.swapaxes(q, 1, 2)
    xTf = jnp.swapaxes(x, 1, 2).reshape(B * N, C_IN)

    i0, i1, i2, w0, w1, w2 = _knn_call(qT, p)
    w0, w1, w2 = (w.reshape(NT, 1, TR) for w in (w0, w1, w2))
    r0, r1, r2 = _build_sc_gather()(i0.reshape(BM), i1.reshape(BM),
                                    i2.reshape(BM), xTf)

    row = pl.BlockSpec((TR, 128), lambda p, i: (jnp.where(p == 0, i, 0), 0))
    w_spec = pl.BlockSpec((1, 1, TR),
                          lambda p, i: (jnp.where(p == 0, i, 0), 0, 0))
    vec = pl.BlockSpec((C1, 1), lambda p, i: (0, 0))
    mat = pl.BlockSpec((C1, C1), lambda p, i: (0, 0))

    h = pl.pallas_call(
        _mlp_body, grid=(3, NT),
        in_specs=[row, row, row, w_spec, w_spec, w_spec,
                  mat, vec, vec, vec, mat, vec, vec, vec],
        out_specs=pl.BlockSpec(
            (1, C2, TR),
            lambda p, i: (jnp.where(p == 2, i // MT, 0), 0,
                          jnp.where(p == 2, i % MT, 0))),
        out_shape=jax.ShapeDtypeStruct((B, C2, M), jnp.float32),
        scratch_shapes=[
            pltpu.VMEM((C1, BM), jnp.float32),
            pltpu.VMEM((C2, BM), jnp.float32),
            pltpu.VMEM((C1, 1), jnp.float32),
            pltpu.VMEM((C1, 1), jnp.float32),
            pltpu.VMEM((C2, 1), jnp.float32),
            pltpu.VMEM((C2, 1), jnp.float32),
        ],
    )(r0, r1, r2, w0, w1, w2, W1, b1[:, None], g1[:, None], be1[:, None],
      W2, b2[:, None], g2[:, None], be2[:, None])

    return (q, h)


# speedup vs baseline: 1.6592x; 1.0084x over previous
import functools

import jax
import jax.numpy as jnp
from jax import lax
from jax.experimental import pallas as pl
from jax.experimental.pallas import tpu as pltpu
from jax.experimental.pallas import tpu_sc as plsc

B, N, M = 2, 2048, 8192
C_IN, C1, C2 = 64, 64, 64
K = 3
TM = 512
NB = M // TM
BM = B * M

NCORES = 2
NSUB = 16
NW = NCORES * NSUB
QPW = BM // NW
CH = 256


_FBIG = 3e38
_G = 8
_O2 = 4096.0
_O3 = 8192.0


def _ins3(a, b, c, v):
    a2 = jnp.minimum(a, v)
    t = jnp.maximum(a, v)
    b2 = jnp.minimum(b, t)
    u = jnp.maximum(b, t)
    c2 = jnp.minimum(c, u)
    return a2, b2, c2


def _fold3(a, b, c):
    g = a.shape[0]
    while g > 1:
        h = g // 2
        a1, b1, c1 = a[:h], b[:h], c[:h]
        for v in (a[h:], b[h:], c[h:]):
            a1, b1, c1 = _ins3(a1, b1, c1, v)
        a, b, c = a1, b1, c1
        g = h
    return a, b, c


def _knn_body(qT_ref, p_ref, i0_ref, i1_ref, i2_ref, w0_ref, w1_ref, w2_ref):
    b = pl.program_id(0)
    qt = qT_ref[0]
    qx, qy, qz = qt[0:1, :], qt[1:2, :], qt[2:3, :]
    pt = p_ref[0]
    px, py, pz = pt[:, 0:1], pt[:, 1:2], pt[:, 2:3]
    dx = px - qx
    dy = py - qy
    dz = pz - qz
    d = dx * dx + dy * dy + dz * dz

    full = functools.partial(jnp.full, (_G, TM), dtype=jnp.float32)
    a, bb, c = full(_FBIG), full(_FBIG), full(_FBIG)
    for s in range(N // _G):
        a, bb, c = _ins3(a, bb, c, d[s * _G:(s + 1) * _G])
    m1, m2, m3 = _fold3(a, bb, c)

    io = lax.broadcasted_iota(jnp.int32, (_G, TM), 0).astype(jnp.float32)
    ja, jb, jc = full(_FBIG), full(_FBIG), full(_FBIG)
    for s in range(N // _G):
        ds = d[s * _G:(s + 1) * _G]
        ios = io + float(s * _G)
        cls = jnp.where(ds == m1, 0.0,
                        jnp.where(ds == m2, _O2,
                                  jnp.where(ds == m3, _O3, _FBIG)))
        ks = ios + cls
        ja, jb, jc = _ins3(ja, jb, jc, ks)
    k1, k2, k3 = _fold3(ja, jb, jc)

    def deco(k):
        return k - jnp.where(k >= _O3, _O3, jnp.where(k >= _O2, _O2, 0.0))

    idxs = [jnp.int32(deco(k)) for k in (k1, k2, k3)]
    ws = [1.0 / jnp.maximum(v, 1e-10) for v in (m1, m2, m3)]
    wsum = ws[0] + ws[1] + ws[2]
    base = b * N
    for i_ref, w_ref, i, w in zip((i0_ref, i1_ref, i2_ref),
                                  (w0_ref, w1_ref, w2_ref), idxs, ws):
        i_ref[...] = (i + base)[None]
        w_ref[...] = (w / wsum)[None]


NCH = QPW // CH


def _sc_gather_body(idx0, idx1, idx2, xT, out0, out1, out2, *scr):
    nj = 3 * NCH
    idx_v = scr[0:nj]
    rows_v = scr[nj:2 * nj]
    isems = scr[2 * nj:3 * nj]
    osems = scr[3 * nj:4 * nj]
    ins = (idx0, idx1, idx2)
    outs = (out0, out1, out2)
    wid = lax.axis_index("s") * NCORES + lax.axis_index("c")
    qb = wid * QPW
    gops = []
    for ch in range(NCH):
        cb = qb + ch * CH
        for k in range(3):
            j = ch * 3 + k
            pltpu.sync_copy(ins[k].at[pl.ds(cb, CH)], idx_v[j])
            gops.append(pltpu.async_copy(xT.at[idx_v[j]], rows_v[j],
                                         isems[j]))
    oops = []
    for ch in range(NCH):
        cb = qb + ch * CH
        for k in range(3):
            j = ch * 3 + k
            gops[j].wait()
            oops.append(pltpu.async_copy(
                rows_v[j], outs[k].at[pl.ds(cb, CH), pl.ds(0, C_IN)],
                osems[j]))
    for op in oops:
        op.wait()


TR = 4096
NT = BM // TR
MT = M // TR


def _bn_cm(u, s_ref, q_ref, g_ref, be_ref):
    mu = s_ref[...] * (1.0 / BM)
    var = q_ref[...] * (1.0 / BM) - mu * mu
    r = (u - mu) * lax.rsqrt(var + 1e-5) * g_ref[...] + be_ref[...]
    return jnp.maximum(r, 0.0)


def _mlp_body(r0, r1, r2, w0, w1, w2, W1, b1, g1, be1, W2, b2, g2, be2,
              out_ref, u1s, u2s, s1, q1, s2, q2):
    p = pl.program_id(0)
    i = pl.program_id(1)
    sl = pl.ds(i * TR, TR)

    @pl.when(jnp.logical_and(p == 0, i == 0))
    def _():
        for r in (s1, q1, s2, q2):
            r[...] = jnp.zeros_like(r)

    @pl.when(p == 0)
    def _():
        Wm = W1[...]
        vs = []
        for r in (r0, r1, r2):
            vs.append(lax.dot_general(Wm, r[...][:, 0:C_IN],
                                      (((1,), (1,)), ((), ())),
                                      preferred_element_type=jnp.float32))
        u = (vs[0] * w0[0] + vs[1] * w1[0] + vs[2] * w2[0]) + b1[...]
        u1s[:, sl] = u
        s1[...] += jnp.sum(u, axis=1, keepdims=True)
        q1[...] += jnp.sum(u * u, axis=1, keepdims=True)

    @pl.when(p == 1)
    def _():
        r = _bn_cm(u1s[:, sl], s1, q1, g1, be1)
        u = lax.dot_general(W2[...], r, (((1,), (0,)), ((), ())),
                            preferred_element_type=jnp.float32) + b2[...]
        u2s[:, sl] = u
        s2[...] += jnp.sum(u, axis=1, keepdims=True)
        q2[...] += jnp.sum(u * u, axis=1, keepdims=True)

    @pl.when(p == 2)
    def _():
        out_ref[0] = _bn_cm(u2s[:, sl], s2, q2, g2, be2)


def _knn_call(qT, p):
    iw_spec = pl.BlockSpec((1, 1, TM), lambda b, i: (b * NB + i, 0, 0))
    return pl.pallas_call(
        _knn_body,
        grid=(B, NB),
        in_specs=[
            pl.BlockSpec((1, 3, TM), lambda b, i: (b, 0, i)),
            pl.BlockSpec((1, N, 3), lambda b, i: (b, 0, 0)),
        ],
        out_specs=[iw_spec] * 6,
        out_shape=[jax.ShapeDtypeStruct((B * NB, 1, TM), jnp.int32)] * 3
        + [jax.ShapeDtypeStruct((B * NB, 1, TM), jnp.float32)] * 3,
    )(qT, p)


@functools.lru_cache(maxsize=1)
def _build_sc_gather():
    @functools.partial(
        pl.kernel,
        out_type=[jax.ShapeDtypeStruct((BM, 128), jnp.float32)] * 3,
        mesh=plsc.VectorSubcoreMesh(core_axis_name="c", subcore_axis_name="s"),
        compiler_params=pltpu.CompilerParams(use_tc_tiling_on_sc=False),
        scratch_types=(
            [pltpu.VMEM((CH,), jnp.int32)] * (3 * NCH)
            + [pltpu.VMEM((CH, C_IN), jnp.float32)] * (3 * NCH)
            + [pltpu.SemaphoreType.DMA] * (6 * NCH)
        ),
    )
    def _sc_gather(idx0, idx1, idx2, xT, out0, out1, out2, *scratch):
        _sc_gather_body(idx0, idx1, idx2, xT, out0, out1, out2, *scratch)

    return _sc_gather


def kernel(p, q, x, W1, b1, g1, be1, W2, b2, g2, be2):
    qT = jnp.swapaxes(q, 1, 2)
    xTf = jnp.swapaxes(x, 1, 2).reshape(B * N, C_IN)

    i0, i1, i2, w0, w1, w2 = _knn_call(qT, p)
    w0, w1, w2 = (w.reshape(NT, 1, TR) for w in (w0, w1, w2))
    r0, r1, r2 = _build_sc_gather()(i0.reshape(BM), i1.reshape(BM),
                                    i2.reshape(BM), xTf)

    row = pl.BlockSpec((TR, 128),
                       lambda p, i: (jnp.where(p == 0, i, NT - 1), 0))
    w_spec = pl.BlockSpec((1, 1, TR),
                          lambda p, i: (jnp.where(p == 0, i, NT - 1), 0, 0))
    vec = pl.BlockSpec((C1, 1), lambda p, i: (0, 0))
    mat = pl.BlockSpec((C1, C1), lambda p, i: (0, 0))

    h = pl.pallas_call(
        _mlp_body, grid=(3, NT),
        in_specs=[row, row, row, w_spec, w_spec, w_spec,
                  mat, vec, vec, vec, mat, vec, vec, vec],
        out_specs=pl.BlockSpec(
            (1, C2, TR),
            lambda p, i: (jnp.where(p == 2, i // MT, 0), 0,
                          jnp.where(p == 2, i % MT, 0))),
        out_shape=jax.ShapeDtypeStruct((B, C2, M), jnp.float32),
        scratch_shapes=[
            pltpu.VMEM((C1, BM), jnp.float32),
            pltpu.VMEM((C2, BM), jnp.float32),
            pltpu.VMEM((C1, 1), jnp.float32),
            pltpu.VMEM((C1, 1), jnp.float32),
            pltpu.VMEM((C2, 1), jnp.float32),
            pltpu.VMEM((C2, 1), jnp.float32),
        ],
    )(r0, r1, r2, w0, w1, w2, W1, b1[:, None], g1[:, None], be1[:, None],
      W2, b2[:, None], g2[:, None], be2[:, None])

    return (q, h)
